# baseline (speedup 1.0000x reference)
import functools

import jax
import jax.numpy as jnp
from jax import lax
from jax.experimental import pallas as pl
from jax.experimental.pallas import tpu as pltpu
from jax.experimental.pallas import tpu_sc as plsc

VOCAB = 1000000
EMBED_DIM = 32
BATCH = 16384

NUM_CORES = 2
NUM_SUBCORES = 16
NUM_WORKERS = NUM_CORES * NUM_SUBCORES
LANES = 16

B_PER_W = BATCH // NUM_WORKERS
CHUNK = 128
N_CHUNK = B_PER_W // CHUNK
NBUF = 2
ROWS_PER_SUPER = 128 // EMBED_DIM


def _scalar_rsqrt(x):
    i = lax.bitcast_convert_type(x, jnp.int32)
    i = 0x5F3759DF - lax.shift_right_logical(i, 1)
    y = lax.bitcast_convert_type(i, jnp.float32)
    for _ in range(3):
        y = y * (1.5 - 0.5 * x * y * y)
    return y


def _tower_body(ids_hbm, table_hbm, out_hbm, idsv, tidv, offv, srbuf, rows, sem):
    wid = lax.axis_index("s") * NUM_CORES + lax.axis_index("c")

    pltpu.sync_copy(ids_hbm.at[pl.ds(wid * 4, 4)], idsv)

    for j in range(4):
        for k in range(8):
            v = idsv[j, pl.ds(k * LANES, LANES)]
            base = j * 128 + k * LANES
            tidv[pl.ds(base, LANES)] = lax.shift_right_logical(v, 2)
            offv[pl.ds(base, LANES)] = lax.shift_left(
                lax.bitwise_and(v, ROWS_PER_SUPER - 1), 5
            )

    def fetch(c, slot):
        return pltpu.async_copy(
            table_hbm.at[tidv.at[pl.ds(c * CHUNK, CHUNK)]],
            srbuf.at[slot],
            sem,
        )

    fetch(0, 0)

    def step(c, carry):
        slot = lax.rem(c, NBUF)
        @pl.when(c + 1 < N_CHUNK)
        def _():
            fetch(c + 1, lax.rem(c + 1, NBUF))

        pltpu.make_async_copy(
            table_hbm.at[tidv.at[pl.ds(c * CHUNK, CHUNK)]],
            srbuf.at[slot],
            sem,
        ).wait()

        for g in range(CHUNK // LANES):
            offs = offv[pl.ds(c * CHUNK + g * LANES, LANES)]
            for r in range(LANES):
                off = lax.squeeze(lax.slice(offs, (r,), (r + 1,)), (0,))
                sr = g * LANES + r
                a = srbuf[slot, sr, pl.ds(off, LANES)]
                b = srbuf[slot, sr, pl.ds(off + LANES, LANES)]
                h = a * a + b * b
                ssq = jnp.sum(h)
                scale = _scalar_rsqrt(jnp.maximum(ssq, 1e-24))
                row = c * CHUNK + sr
                t, s = row // 8, row % 8
                rows[t, s, pl.ds(0, LANES)] = a * scale
                rows[t, s, pl.ds(LANES, LANES)] = b * scale
        return carry

    lax.fori_loop(0, N_CHUNK, step, 0)

    pltpu.sync_copy(rows, out_hbm.at[pl.ds(wid * (B_PER_W // 8), B_PER_W // 8)])


_tower = functools.partial(
    pl.kernel,
    out_type=jax.ShapeDtypeStruct((BATCH // 8, 8, EMBED_DIM), jnp.float32),
    mesh=plsc.VectorSubcoreMesh(core_axis_name="c", subcore_axis_name="s"),
    compiler_params=pltpu.CompilerParams(needs_layout_passes=False),
    scratch_types=[
        pltpu.VMEM((4, 128), jnp.int32),
        pltpu.VMEM((B_PER_W,), jnp.int32),
        pltpu.VMEM((B_PER_W,), jnp.int32),
        pltpu.VMEM((NBUF, CHUNK, 128), jnp.float32),
        pltpu.VMEM((B_PER_W // 8, 8, EMBED_DIM), jnp.float32),
        pltpu.SemaphoreType.DMA,
    ],
)(_tower_body)


def kernel(item_ids, embedding_table):
    ids = item_ids.astype(jnp.int32).reshape(128, 128)
    table2 = embedding_table.reshape(VOCAB // ROWS_PER_SUPER, 128)
    out3 = _tower(ids, table2)
    return out3.reshape(BATCH, EMBED_DIM)


# speedup vs baseline: 2.7405x; 2.7405x over previous
import functools

import jax
import jax.numpy as jnp
from jax import lax
from jax.experimental import pallas as pl
from jax.experimental.pallas import tpu as pltpu
from jax.experimental.pallas import tpu_sc as plsc

VOCAB = 1000000
EMBED_DIM = 32
BATCH = 16384

NUM_CORES = 2
NUM_SUBCORES = 16
NUM_WORKERS = NUM_CORES * NUM_SUBCORES
LANES = 16

B_PER_W = BATCH // NUM_WORKERS
GROUP = 16
N_GROUP = B_PER_W // GROUP


def _scalar_rsqrt(x):
    i = lax.bitcast_convert_type(x, jnp.int32)
    i = 0x5F3759DF - lax.shift_right_logical(i, 1)
    y = lax.bitcast_convert_type(i, jnp.float32)
    for _ in range(3):
        y = y * (1.5 - 0.5 * x * y * y)
    return y


def _tower_body(ids_hbm, table_hbm, out_hbm, idsv, rows, sem):
    wid = lax.axis_index("s") * NUM_CORES + lax.axis_index("c")

    pltpu.sync_copy(ids_hbm.at[pl.ds(wid * 4, 4)], idsv)

    for j in range(4):
        for k in range(8):
            v = idsv[j, pl.ds(k * LANES, LANES)]
            tid16 = lax.shift_right_logical(v, 3)
            sub16 = lax.bitwise_and(v, 7)
            for r in range(LANES):
                tid = lax.squeeze(lax.slice(tid16, (r,), (r + 1,)), (0,))
                sub = lax.squeeze(lax.slice(sub16, (r,), (r + 1,)), (0,))
                row = j * 128 + k * LANES + r
                pltpu.async_copy(
                    table_hbm.at[tid, sub],
                    rows.at[row // 8, row % 8],
                    sem,
                )

    def step(g, carry):
        for r in range(GROUP):
            pltpu.make_async_copy(
                table_hbm.at[0, 0], rows.at[0, 0], sem
            ).wait()
        for r in range(GROUP):
            t = g * 2 + r // 8
            s = r % 8
            a = rows[t, s, pl.ds(0, LANES)]
            b = rows[t, s, pl.ds(LANES, LANES)]
            h = a * a + b * b
            ssq = jnp.sum(h)
            scale = _scalar_rsqrt(jnp.maximum(ssq, 1e-24))
            rows[t, s, pl.ds(0, LANES)] = a * scale
            rows[t, s, pl.ds(LANES, LANES)] = b * scale
        return carry

    lax.fori_loop(0, N_GROUP, step, 0)

    pltpu.sync_copy(rows, out_hbm.at[pl.ds(wid * (B_PER_W // 8), B_PER_W // 8)])


_tower = functools.partial(
    pl.kernel,
    out_type=jax.ShapeDtypeStruct((BATCH // 8, 8, EMBED_DIM), jnp.float32),
    mesh=plsc.VectorSubcoreMesh(core_axis_name="c", subcore_axis_name="s"),
    compiler_params=pltpu.CompilerParams(needs_layout_passes=False),
    scratch_types=[
        pltpu.VMEM((4, 128), jnp.int32),
        pltpu.VMEM((B_PER_W // 8, 8, EMBED_DIM), jnp.float32),
        pltpu.SemaphoreType.DMA,
    ],
)(_tower_body)


def kernel(item_ids, embedding_table):
    ids = item_ids.astype(jnp.int32).reshape(128, 128)
    table3 = embedding_table.reshape(VOCAB // 8, 8, EMBED_DIM)
    out3 = _tower(ids, table3)
    return out3.reshape(BATCH, EMBED_DIM)


# speedup vs baseline: 2.7490x; 1.0031x over previous
import functools

import jax
import jax.numpy as jnp
from jax import lax
from jax.experimental import pallas as pl
from jax.experimental.pallas import tpu as pltpu
from jax.experimental.pallas import tpu_sc as plsc

VOCAB = 1000000
EMBED_DIM = 32
BATCH = 16384

NUM_CORES = 2
NUM_SUBCORES = 16
NUM_WORKERS = NUM_CORES * NUM_SUBCORES
LANES = 16

B_PER_W = BATCH // NUM_WORKERS
GROUP = 16
N_GROUP = B_PER_W // GROUP
DEPTH = 4


def _scalar_rsqrt(x):
    i = lax.bitcast_convert_type(x, jnp.int32)
    i = 0x5F3759DF - lax.shift_right_logical(i, 1)
    y = lax.bitcast_convert_type(i, jnp.float32)
    for _ in range(3):
        y = y * (1.5 - 0.5 * x * y * y)
    return y


def _tower_body(ids_hbm, table_hbm, out_hbm, idsv, rows, *sems):
    wid = lax.axis_index("s") * NUM_CORES + lax.axis_index("c")

    pltpu.sync_copy(ids_hbm.at[pl.ds(wid * 4, 4)], idsv)

    def enqueue_group(g, sem):
        j = g // 8
        k = lax.rem(g, 8) if not isinstance(g, int) else g % 8
        v = idsv[j, pl.ds(k * LANES, LANES)]
        tid16 = lax.shift_right_logical(v, 3)
        sub16 = lax.bitwise_and(v, 7)
        for r in range(GROUP):
            tid = lax.squeeze(lax.slice(tid16, (r,), (r + 1,)), (0,))
            sub = lax.squeeze(lax.slice(sub16, (r,), (r + 1,)), (0,))
            t = g * 2 + r // 8
            pltpu.async_copy(
                table_hbm.at[tid, sub],
                rows.at[t, r % 8],
                sem,
            )

    for g in range(DEPTH):
        enqueue_group(g, sems[g])

    def outer(o, carry):
        for s in range(DEPTH):
            g = o * DEPTH + s
            for r in range(GROUP):
                pltpu.make_async_copy(
                    table_hbm.at[0, 0], rows.at[0, 0], sems[s]
                ).wait()
            for r in range(GROUP):
                t = g * 2 + r // 8
                a = rows[t, r % 8, pl.ds(0, LANES)]
                b = rows[t, r % 8, pl.ds(LANES, LANES)]
                h = a * a + b * b
                ssq = jnp.sum(h)
                scale = _scalar_rsqrt(jnp.maximum(ssq, 1e-24))
                rows[t, r % 8, pl.ds(0, LANES)] = a * scale
                rows[t, r % 8, pl.ds(LANES, LANES)] = b * scale
            @pl.when(o < N_GROUP // DEPTH - 1)
            def _():
                enqueue_group(g + DEPTH, sems[s])
        return carry

    lax.fori_loop(0, N_GROUP // DEPTH, outer, 0)

    pltpu.sync_copy(rows, out_hbm.at[pl.ds(wid * (B_PER_W // 8), B_PER_W // 8)])


_tower = functools.partial(
    pl.kernel,
    out_type=jax.ShapeDtypeStruct((BATCH // 8, 8, EMBED_DIM), jnp.float32),
    mesh=plsc.VectorSubcoreMesh(core_axis_name="c", subcore_axis_name="s"),
    compiler_params=pltpu.CompilerParams(needs_layout_passes=False),
    scratch_types=[
        pltpu.VMEM((4, 128), jnp.int32),
        pltpu.VMEM((B_PER_W // 8, 8, EMBED_DIM), jnp.float32),
    ] + [pltpu.SemaphoreType.DMA] * DEPTH,
)(_tower_body)


def kernel(item_ids, embedding_table):
    ids = item_ids.astype(jnp.int32).reshape(128, 128)
    table3 = embedding_table.reshape(VOCAB // 8, 8, EMBED_DIM)
    out3 = _tower(ids, table3)
    return out3.reshape(BATCH, EMBED_DIM)
